# Initial kernel scaffold; baseline (speedup 1.0000x reference)
import jax
import jax.numpy as jnp
from jax.experimental import pallas as pl


def kernel(x, table, W, b):
    raise NotImplementedError("write your pallas kernel here")


# speedup vs baseline: 3.4559x; 3.4559x over previous
import functools

import jax
import jax.numpy as jnp
from jax import lax
from jax.experimental import pallas as pl
from jax.experimental.pallas import tpu as pltpu
from jax.experimental.pallas import tpu_sc as plsc

B = 4096
S = 200
D = 64
NUM_CLASSES = 128

NC = 2
NS = 16
NW = NC * NS
B_PER_W = B // NW
HALF = 100
HALF_PAD = 104
G_PER_W = 2 * B_PER_W
NBUF = 4
INV_S = 1.0 / S


def _sc_pool_body(x_hbm, table_hbm, out_hbm,
                  ib0, ib1, ib2, ib3, buf0, buf1, buf2, buf3,
                  pooled_v, sem0, sem1, sem2, sem3,
                  isem0, isem1, isem2, isem3):
    bufs = (buf0, buf1, buf2, buf3)
    ibs = (ib0, ib1, ib2, ib3)
    sems = (sem0, sem1, sem2, sem3)
    isems = (isem0, isem1, isem2, isem3)

    wid = lax.axis_index("s") * NC + lax.axis_index("c")
    idx_base = wid * G_PER_W

    def fetch_idx(g, k):
        pltpu.async_copy(x_hbm.at[idx_base + g], ibs[k], isems[k])

    def fire(k):
        pltpu.make_async_copy(x_hbm.at[idx_base], ibs[k], isems[k]).wait()
        pltpu.async_copy(table_hbm.at[ibs[k]], bufs[k], sems[k])

    def wait(k):
        pltpu.make_async_copy(table_hbm.at[ibs[k]], bufs[k], sems[k]).wait()

    for k in range(NBUF):
        fetch_idx(k, k)
    for k in range(NBUF):
        fire(k)

    def accumulate(buf):
        def body(i, acc):
            a0, a1, a2, a3 = acc
            for u in range(4):
                r = i * 4 + u
                a0 = a0 + buf[r, pl.ds(0, 16)]
                a1 = a1 + buf[r, pl.ds(16, 16)]
                a2 = a2 + buf[r, pl.ds(32, 16)]
                a3 = a3 + buf[r, pl.ds(48, 16)]
            return a0, a1, a2, a3
        zero = jnp.zeros((16,), jnp.float32)
        return lax.fori_loop(0, HALF // 4, body, (zero, zero, zero, zero))

    def outer(j, carry):
        g_base = j * NBUF
        for k in range(NBUF):
            g = g_base + k
            wait(k)

            @pl.when(j < (G_PER_W // NBUF) - 1)
            def _():
                fetch_idx(g + NBUF, k)

            a0, a1, a2, a3 = accumulate(bufs[k])
            item = (g_base + k) // 2
            if k % 2 == 0:
                pooled_v[item, pl.ds(0, 16)] = a0
                pooled_v[item, pl.ds(16, 16)] = a1
                pooled_v[item, pl.ds(32, 16)] = a2
                pooled_v[item, pl.ds(48, 16)] = a3
            else:
                pooled_v[item, pl.ds(0, 16)] = (
                    pooled_v[item, pl.ds(0, 16)] + a0) * INV_S
                pooled_v[item, pl.ds(16, 16)] = (
                    pooled_v[item, pl.ds(16, 16)] + a1) * INV_S
                pooled_v[item, pl.ds(32, 16)] = (
                    pooled_v[item, pl.ds(32, 16)] + a2) * INV_S
                pooled_v[item, pl.ds(48, 16)] = (
                    pooled_v[item, pl.ds(48, 16)] + a3) * INV_S

            @pl.when(j < (G_PER_W // NBUF) - 1)
            def _():
                fire(k)
        return carry

    lax.fori_loop(0, G_PER_W // NBUF, outer, 0)

    pltpu.sync_copy(pooled_v, out_hbm.at[pl.ds(wid * B_PER_W, B_PER_W)])


@functools.lru_cache(maxsize=None)
def _make_sc_pool():
    return pl.kernel(
        _sc_pool_body,
        out_type=jax.ShapeDtypeStruct((B, D), jnp.float32),
        mesh=plsc.VectorSubcoreMesh(core_axis_name="c", subcore_axis_name="s",
                                    num_cores=NC, num_subcores=NS),
        compiler_params=pltpu.CompilerParams(use_tc_tiling_on_sc=False),
        scratch_types=[
            pltpu.VMEM((HALF_PAD,), jnp.int32),
            pltpu.VMEM((HALF_PAD,), jnp.int32),
            pltpu.VMEM((HALF_PAD,), jnp.int32),
            pltpu.VMEM((HALF_PAD,), jnp.int32),
            pltpu.VMEM((HALF_PAD, D), jnp.float32),
            pltpu.VMEM((HALF_PAD, D), jnp.float32),
            pltpu.VMEM((HALF_PAD, D), jnp.float32),
            pltpu.VMEM((HALF_PAD, D), jnp.float32),
            pltpu.VMEM((B_PER_W, D), jnp.float32),
            pltpu.SemaphoreType.DMA,
            pltpu.SemaphoreType.DMA,
            pltpu.SemaphoreType.DMA,
            pltpu.SemaphoreType.DMA,
            pltpu.SemaphoreType.DMA,
            pltpu.SemaphoreType.DMA,
            pltpu.SemaphoreType.DMA,
            pltpu.SemaphoreType.DMA,
        ],
    )


def _mm_body(p_ref, w_ref, b_ref, o_ref):
    o_ref[...] = jnp.dot(p_ref[...], w_ref[...],
                         preferred_element_type=jnp.float32) + b_ref[...]


def _tc_matmul(pooled, W, b):
    blk = 512
    return pl.pallas_call(
        _mm_body,
        grid=(B // blk,),
        in_specs=[
            pl.BlockSpec((blk, D), lambda i: (i, 0)),
            pl.BlockSpec((D, NUM_CLASSES), lambda i: (0, 0)),
            pl.BlockSpec((NUM_CLASSES,), lambda i: (0,)),
        ],
        out_specs=pl.BlockSpec((blk, NUM_CLASSES), lambda i: (i, 0)),
        out_shape=jax.ShapeDtypeStruct((B, NUM_CLASSES), jnp.float32),
    )(pooled, W, b)


@jax.jit
def kernel(x, table, W, b):
    xr = x.astype(jnp.int32).reshape(B, 2, HALF)
    xp = jnp.pad(xr, ((0, 0), (0, 0), (0, HALF_PAD - HALF)))
    xp = xp.reshape(B * 2, HALF_PAD)
    pooled = _make_sc_pool()(xp, table)
    return _tc_matmul(pooled, W, b)


# speedup vs baseline: 13.2453x; 3.8327x over previous
import functools

import jax
import jax.numpy as jnp
from jax import lax
from jax.experimental import pallas as pl
from jax.experimental.pallas import tpu as pltpu
from jax.experimental.pallas import tpu_sc as plsc

B = 4096
S = 200
D = 64
NUM_CLASSES = 128

NC = 2
NS = 16
NW = NC * NS
B_PER_W = B // NW
SPLIT_A = 104
SPLIT_B = S - SPLIT_A
G_PER_W = 2 * B_PER_W
NBUF = 8
INV_S = 1.0 / S

_LENS = tuple(SPLIT_A if k % 2 == 0 else SPLIT_B for k in range(NBUF))


def _sc_pool_body(x_hbm, table_hbm, out_hbm, *refs):
    ibs = refs[:NBUF]
    bufs = refs[NBUF:2 * NBUF]
    pooled_v = refs[2 * NBUF]
    sems = refs[2 * NBUF + 1:2 * NBUF + 1 + NBUF]
    isems = refs[2 * NBUF + 1 + NBUF:]

    wid = lax.axis_index("s") * NC + lax.axis_index("c")
    flat_base = wid * B_PER_W * S

    def fetch_idx(g, k):
        off = flat_base + (g // 2) * S + (k % 2) * SPLIT_A
        pltpu.async_copy(x_hbm.at[pl.ds(off, _LENS[k])], ibs[k], isems[k])

    def fire(k):
        pltpu.make_async_copy(x_hbm.at[pl.ds(0, _LENS[k])], ibs[k],
                              isems[k]).wait()
        pltpu.async_copy(table_hbm.at[ibs[k]], bufs[k], sems[k])

    def wait(k):
        pltpu.make_async_copy(table_hbm.at[ibs[k]], bufs[k], sems[k]).wait()

    for k in range(NBUF):
        fetch_idx(k, k)
    for k in range(NBUF):
        fire(k)

    def accumulate(buf, n_rows):
        def body(i, acc):
            a0, a1, a2, a3 = acc
            for u in range(4):
                r = i * 4 + u
                a0 = a0 + buf[r, pl.ds(0, 16)]
                a1 = a1 + buf[r, pl.ds(16, 16)]
                a2 = a2 + buf[r, pl.ds(32, 16)]
                a3 = a3 + buf[r, pl.ds(48, 16)]
            return a0, a1, a2, a3
        zero = jnp.zeros((16,), jnp.float32)
        return lax.fori_loop(0, n_rows // 4, body, (zero, zero, zero, zero))

    def outer(j, carry):
        g_base = j * NBUF
        for k in range(NBUF):
            g = g_base + k
            wait(k)

            @pl.when(j < (G_PER_W // NBUF) - 1)
            def _():
                fetch_idx(g + NBUF, k)

            a0, a1, a2, a3 = accumulate(bufs[k], _LENS[k])
            item = (g_base + k) // 2
            if k % 2 == 0:
                pooled_v[item, pl.ds(0, 16)] = a0
                pooled_v[item, pl.ds(16, 16)] = a1
                pooled_v[item, pl.ds(32, 16)] = a2
                pooled_v[item, pl.ds(48, 16)] = a3
            else:
                pooled_v[item, pl.ds(0, 16)] = (
                    pooled_v[item, pl.ds(0, 16)] + a0) * INV_S
                pooled_v[item, pl.ds(16, 16)] = (
                    pooled_v[item, pl.ds(16, 16)] + a1) * INV_S
                pooled_v[item, pl.ds(32, 16)] = (
                    pooled_v[item, pl.ds(32, 16)] + a2) * INV_S
                pooled_v[item, pl.ds(48, 16)] = (
                    pooled_v[item, pl.ds(48, 16)] + a3) * INV_S

            @pl.when(j < (G_PER_W // NBUF) - 1)
            def _():
                fire(k)
        return carry

    lax.fori_loop(0, G_PER_W // NBUF, outer, 0)

    pltpu.sync_copy(pooled_v, out_hbm.at[pl.ds(wid * B_PER_W, B_PER_W)])


@functools.lru_cache(maxsize=None)
def _make_sc_pool():
    return pl.kernel(
        _sc_pool_body,
        out_type=jax.ShapeDtypeStruct((B, D), jnp.float32),
        mesh=plsc.VectorSubcoreMesh(core_axis_name="c", subcore_axis_name="s",
                                    num_cores=NC, num_subcores=NS),
        compiler_params=pltpu.CompilerParams(use_tc_tiling_on_sc=False),
        scratch_types=(
            [pltpu.VMEM((_LENS[k],), jnp.int32) for k in range(NBUF)]
            + [pltpu.VMEM((_LENS[k], D), jnp.float32) for k in range(NBUF)]
            + [pltpu.VMEM((B_PER_W, D), jnp.float32)]
            + [pltpu.SemaphoreType.DMA] * (2 * NBUF)
        ),
    )


def _mm_body(p_ref, w_ref, b_ref, o_ref):
    o_ref[...] = jnp.dot(p_ref[...], w_ref[...],
                         preferred_element_type=jnp.float32) + b_ref[...]


def _tc_matmul(pooled, W, b):
    blk = 512
    return pl.pallas_call(
        _mm_body,
        grid=(B // blk,),
        in_specs=[
            pl.BlockSpec((blk, D), lambda i: (i, 0)),
            pl.BlockSpec((D, NUM_CLASSES), lambda i: (0, 0)),
            pl.BlockSpec((NUM_CLASSES,), lambda i: (0,)),
        ],
        out_specs=pl.BlockSpec((blk, NUM_CLASSES), lambda i: (i, 0)),
        out_shape=jax.ShapeDtypeStruct((B, NUM_CLASSES), jnp.float32),
    )(pooled, W, b)


@jax.jit
def kernel(x, table, W, b):
    xf = x.astype(jnp.int32).reshape(B * S)
    pooled = _make_sc_pool()(xf, table)
    return _tc_matmul(pooled, W, b)
